# Initial kernel scaffold; baseline (speedup 1.0000x reference)
import jax
import jax.numpy as jnp
from jax.experimental import pallas as pl


def kernel(epoch, z1, z2):
    raise NotImplementedError("write your pallas kernel here")


# speedup vs baseline: 239.4726x; 239.4726x over previous
import jax
import jax.numpy as jnp
from jax.experimental import pallas as pl
from jax.experimental.pallas import tpu as pltpu

_N = 10000
_D = 128
_K = 1024
_LAM = 0.5
_RB = 400
_G = _N // _RB
_CB = 500
_NC = _N // _CB
_NT = 7


def _lane_pack(pairs):
    lane = jax.lax.broadcasted_iota(jnp.int32, (1, 128), 1)
    row = jnp.zeros((1, 128), dtype=jnp.float32)
    for idx, val in pairs:
        row = jnp.where(lane == idx, val, row)
    return row


def _softplus(x):
    return jnp.maximum(x, 0.0) + jnp.log1p(jnp.exp(-jnp.abs(x)))


def _chunk_dot(a, z2_ref, c):
    zb = z2_ref[pl.ds(c * _CB, _CB), :]
    return jax.lax.dot_general(
        a, zb, (((1,), (1,)), ((), ())),
        preferred_element_type=jnp.float32)


def _pass_a_kernel(z1_ref, z2_ref, out_ref):
    i = pl.program_id(0)
    a = z1_ref[...]

    def body(c, carry):
        mx, mn = carry
        s = _chunk_dot(a, z2_ref, c)
        return jnp.maximum(mx, jnp.max(s)), jnp.minimum(mn, jnp.min(s))

    big = jnp.float32(3.4e38)
    mx, mn = jax.lax.fori_loop(0, _NC, body, (-big, big))

    zb = z2_ref[pl.ds(i * _RB, _RB), :]
    d = jnp.sum(a * zb, axis=1)
    out_ref[...] = _lane_pack([
        (0, mx),
        (1, mn),
        (2, jnp.sum(_softplus(-d))),
        (3, jnp.sum((a - zb) ** 2)),
    ])[None]


def _pass_b_kernel(thr_ref, z1_ref, z2_ref, out_ref):
    a = z1_ref[...]
    ts = [thr_ref[0, j] for j in range(_NT)]

    def body(c, carry):
        s = _chunk_dot(a, z2_ref, c)
        new = []
        for j in range(_NT):
            new.append(carry[2 * j] + jnp.sum(jnp.where(s > ts[j], 1.0, 0.0)))
            new.append(carry[2 * j + 1]
                       + jnp.sum(jnp.where(s < ts[j], 1.0, 0.0)))
        return tuple(new)

    init = tuple(jnp.float32(0.0) for _ in range(2 * _NT))
    res = jax.lax.fori_loop(0, _NC, body, init)
    pairs = [(j, res[2 * j]) for j in range(_NT)]
    pairs += [(_NT + j, res[2 * j + 1]) for j in range(_NT)]
    out_ref[...] = _lane_pack(pairs)[None]


def _pass_c_kernel(par_ref, z1_ref, z2_ref, out_ref):
    a = z1_ref[...]
    u_hi = par_ref[0, 0]
    l_lo = par_ref[0, 1]

    def body(c, carry):
        cnt_t, sum_t, cnt_b, sum_b = carry
        s = _chunk_dot(a, z2_ref, c)
        m_t = s > u_hi
        m_b = s < l_lo
        cnt_t = cnt_t + jnp.sum(jnp.where(m_t, 1.0, 0.0))
        sum_t = sum_t + jnp.sum(jnp.where(m_t, _softplus(-s), 0.0))
        cnt_b = cnt_b + jnp.sum(jnp.where(m_b, 1.0, 0.0))
        sum_b = sum_b + jnp.sum(jnp.where(m_b, _softplus(s), 0.0))
        return cnt_t, sum_t, cnt_b, sum_b

    z = jnp.float32(0.0)
    cnt_t, sum_t, cnt_b, sum_b = jax.lax.fori_loop(0, _NC, body, (z, z, z, z))
    out_ref[...] = _lane_pack([
        (0, cnt_t), (1, sum_t), (2, cnt_b), (3, sum_b),
    ])[None]


_Z1_SPEC = pl.BlockSpec((_RB, _D), lambda i: (i, 0))
_Z2_SPEC = pl.BlockSpec((_N, _D), lambda i: (0, 0))
_OUT_SPEC = pl.BlockSpec((1, 1, 128), lambda i: (i, 0, 0))
_OUT_SHAPE = jax.ShapeDtypeStruct((_G, 1, 128), jnp.float32)
_SMEM_SPEC = pl.BlockSpec(memory_space=pltpu.SMEM)


def kernel(epoch, z1, z2):
    del epoch
    z1 = z1.astype(jnp.float32)
    z2 = z2.astype(jnp.float32)

    pa = pl.pallas_call(
        _pass_a_kernel,
        grid=(_G,),
        in_specs=[_Z1_SPEC, _Z2_SPEC],
        out_specs=_OUT_SPEC,
        out_shape=_OUT_SHAPE,
    )(z1, z2)
    mx = jnp.max(pa[:, 0, 0])
    mn = jnp.min(pa[:, 0, 1])
    sum_diag = jnp.sum(pa[:, 0, 2])
    sum_sq = jnp.sum(pa[:, 0, 3])

    js = jnp.arange(1, _NT + 1, dtype=jnp.float32)
    thr = mn + (mx - mn) * js / (_NT + 1.0)
    thr_in = thr[None, :]

    pb = pl.pallas_call(
        _pass_b_kernel,
        grid=(_G,),
        in_specs=[_SMEM_SPEC, _Z1_SPEC, _Z2_SPEC],
        out_specs=_OUT_SPEC,
        out_shape=_OUT_SHAPE,
    )(thr_in, z1, z2)
    pb = jnp.sum(pb[:, 0, :], axis=0)
    cnt_gt = pb[:_NT]
    cnt_lt = pb[_NT:2 * _NT]

    kf = jnp.float32(_K)
    m_hi = cnt_gt >= kf
    l_hi = jnp.maximum(mn, jnp.max(jnp.where(m_hi, thr, -jnp.inf)))
    u_hi = jnp.minimum(mx, jnp.min(jnp.where(~m_hi, thr, jnp.inf)))
    m_lo = cnt_lt >= kf
    l_lo = jnp.maximum(mn, jnp.max(jnp.where(~m_lo, thr, -jnp.inf)))
    u_lo = jnp.minimum(mx, jnp.min(jnp.where(m_lo, thr, jnp.inf)))

    par = jnp.stack([u_hi, l_lo])[None, :]

    pc = pl.pallas_call(
        _pass_c_kernel,
        grid=(_G,),
        in_specs=[_SMEM_SPEC, _Z1_SPEC, _Z2_SPEC],
        out_specs=_OUT_SPEC,
        out_shape=_OUT_SHAPE,
    )(par, z1, z2)
    pc = jnp.sum(pc[:, 0, :], axis=0)
    cnt_top, sum_top, cnt_bot, sum_bot = pc[0], pc[1], pc[2], pc[3]

    v_hi = 0.5 * (l_hi + u_hi)
    v_lo = 0.5 * (l_lo + u_lo)
    top_term = sum_top + (kf - cnt_top) * _softplus(-v_hi)
    bot_term = sum_bot + (kf - cnt_bot) * _softplus(v_lo)

    n_logits = jnp.float32(_N + 2 * _K)
    bce = (sum_diag + top_term + bot_term) / n_logits
    mse = sum_sq / jnp.float32(_N * _D)
    return bce + _LAM * mse * jnp.float32(_N)


# speedup vs baseline: 334.4276x; 1.3965x over previous
import jax
import jax.numpy as jnp
from jax.experimental import pallas as pl
from jax.experimental.pallas import tpu as pltpu

_N = 10000
_D = 128
_K = 1024
_LAM = 0.5
_RB = 400
_G = _N // _RB
_CB = 1000
_NC = _N // _CB
_NT = 7


def _lane_pack(pairs):
    lane = jax.lax.broadcasted_iota(jnp.int32, (1, 128), 1)
    row = jnp.zeros((1, 128), dtype=jnp.float32)
    for idx, val in pairs:
        row = jnp.where(lane == idx, val, row)
    return row


def _softplus(x):
    return jnp.maximum(x, 0.0) + jnp.log1p(jnp.exp(-jnp.abs(x)))


def _chunk_dot(a, z2_ref, c):
    zb = z2_ref[pl.ds(c * _CB, _CB), :]
    return jax.lax.dot_general(
        a, zb, (((1,), (1,)), ((), ())),
        preferred_element_type=jnp.float32)


def _pass_n_kernel(z1_ref, z2_ref, out_ref):
    a = z1_ref[...]
    zb = z2_ref[pl.ds(pl.program_id(0) * _RB, _RB), :]
    d = jnp.sum(a * zb, axis=1)
    out_ref[...] = _lane_pack([
        (0, jnp.max(jnp.sum(a * a, axis=1))),
        (1, jnp.max(jnp.sum(zb * zb, axis=1))),
        (2, jnp.sum(_softplus(-d))),
        (3, jnp.sum((a - zb) ** 2)),
    ])[None]


def _pass_b_kernel(thr_ref, z1_ref, z2_ref, out_ref):
    a = z1_ref[...]
    ts = [thr_ref[0, j] for j in range(_NT)]

    def body(c, carry):
        s = _chunk_dot(a, z2_ref, c)
        new = []
        for j in range(_NT):
            new.append(carry[2 * j] + jnp.sum(jnp.where(s > ts[j], 1.0, 0.0)))
            new.append(carry[2 * j + 1]
                       + jnp.sum(jnp.where(s < ts[j], 1.0, 0.0)))
        return tuple(new)

    init = tuple(jnp.float32(0.0) for _ in range(2 * _NT))
    res = jax.lax.fori_loop(0, _NC, body, init)
    pairs = [(j, res[2 * j]) for j in range(_NT)]
    pairs += [(_NT + j, res[2 * j + 1]) for j in range(_NT)]
    out_ref[...] = _lane_pack(pairs)[None]


def _pass_c_kernel(par_ref, z1_ref, z2_ref, out_ref):
    a = z1_ref[...]
    u_hi = par_ref[0, 0]
    l_lo = par_ref[0, 1]

    def body(c, carry):
        cnt_t, sum_t, cnt_b, sum_b = carry
        s = _chunk_dot(a, z2_ref, c)
        m_t = s > u_hi
        m_b = s < l_lo
        cnt_t = cnt_t + jnp.sum(jnp.where(m_t, 1.0, 0.0))
        sum_t = sum_t + jnp.sum(jnp.where(m_t, _softplus(-s), 0.0))
        cnt_b = cnt_b + jnp.sum(jnp.where(m_b, 1.0, 0.0))
        sum_b = sum_b + jnp.sum(jnp.where(m_b, _softplus(s), 0.0))
        return cnt_t, sum_t, cnt_b, sum_b

    z = jnp.float32(0.0)
    cnt_t, sum_t, cnt_b, sum_b = jax.lax.fori_loop(0, _NC, body, (z, z, z, z))
    out_ref[...] = _lane_pack([
        (0, cnt_t), (1, sum_t), (2, cnt_b), (3, sum_b),
    ])[None]


_Z1F_SPEC = pl.BlockSpec((_RB, _D), lambda i: (i, 0))
_Z2F_SPEC = pl.BlockSpec((_N, _D), lambda i: (0, 0))
_OUT_SPEC = pl.BlockSpec((1, 1, 128), lambda i: (i, 0, 0))
_OUT_SHAPE = jax.ShapeDtypeStruct((_G, 1, 128), jnp.float32)
_SMEM_SPEC = pl.BlockSpec(memory_space=pltpu.SMEM)


def kernel(epoch, z1, z2):
    del epoch
    z1 = z1.astype(jnp.float32)
    z2 = z2.astype(jnp.float32)
    z1b = z1.astype(jnp.bfloat16)
    z2b = z2.astype(jnp.bfloat16)

    pn = pl.pallas_call(
        _pass_n_kernel,
        grid=(_G,),
        in_specs=[_Z1F_SPEC, _Z2F_SPEC],
        out_specs=_OUT_SPEC,
        out_shape=_OUT_SHAPE,
    )(z1, z2)
    bound = jnp.sqrt(jnp.max(pn[:, 0, 0])) * jnp.sqrt(jnp.max(pn[:, 0, 1]))
    sum_diag = jnp.sum(pn[:, 0, 2])
    sum_sq = jnp.sum(pn[:, 0, 3])

    js = jnp.arange(1, _NT + 1, dtype=jnp.float32)
    thr = -bound + 2.0 * bound * js / (_NT + 1.0)
    thr_in = thr[None, :]

    pb = pl.pallas_call(
        _pass_b_kernel,
        grid=(_G,),
        in_specs=[_SMEM_SPEC, _Z1F_SPEC, _Z2F_SPEC],
        out_specs=_OUT_SPEC,
        out_shape=_OUT_SHAPE,
    )(thr_in, z1b, z2b)
    pb = jnp.sum(pb[:, 0, :], axis=0)
    cnt_gt = pb[:_NT]
    cnt_lt = pb[_NT:2 * _NT]

    kf = jnp.float32(_K)
    m_hi = cnt_gt >= kf
    l_hi = jnp.maximum(-bound, jnp.max(jnp.where(m_hi, thr, -jnp.inf)))
    u_hi = jnp.minimum(bound, jnp.min(jnp.where(~m_hi, thr, jnp.inf)))
    m_lo = cnt_lt >= kf
    l_lo = jnp.maximum(-bound, jnp.max(jnp.where(~m_lo, thr, -jnp.inf)))
    u_lo = jnp.minimum(bound, jnp.min(jnp.where(m_lo, thr, jnp.inf)))

    par = jnp.stack([u_hi, l_lo])[None, :]

    pc = pl.pallas_call(
        _pass_c_kernel,
        grid=(_G,),
        in_specs=[_SMEM_SPEC, _Z1F_SPEC, _Z2F_SPEC],
        out_specs=_OUT_SPEC,
        out_shape=_OUT_SHAPE,
    )(par, z1b, z2b)
    pc = jnp.sum(pc[:, 0, :], axis=0)
    cnt_top, sum_top, cnt_bot, sum_bot = pc[0], pc[1], pc[2], pc[3]

    v_hi = 0.5 * (l_hi + u_hi)
    v_lo = 0.5 * (l_lo + u_lo)
    top_term = sum_top + (kf - cnt_top) * _softplus(-v_hi)
    bot_term = sum_bot + (kf - cnt_bot) * _softplus(v_lo)

    n_logits = jnp.float32(_N + 2 * _K)
    bce = (sum_diag + top_term + bot_term) / n_logits
    mse = sum_sq / jnp.float32(_N * _D)
    return bce + _LAM * mse * jnp.float32(_N)


# speedup vs baseline: 626.2604x; 1.8726x over previous
import jax
import jax.numpy as jnp
from jax.experimental import pallas as pl
from jax.experimental.pallas import tpu as pltpu

_N = 10000
_D = 128
_K = 1024
_LAM = 0.5
_RB = 400
_G = _N // _RB
_CB = 1000
_NC = _N // _CB
_NT = 7


def _lane_pack(pairs):
    lane = jax.lax.broadcasted_iota(jnp.int32, (1, 128), 1)
    row = jnp.zeros((1, 128), dtype=jnp.float32)
    for idx, val in pairs:
        row = jnp.where(lane == idx, val, row)
    return row


def _softplus(x):
    return jnp.maximum(x, 0.0) + jnp.log1p(jnp.exp(-jnp.abs(x)))


def _chunk_dot(a, z2_ref, c):
    zb = z2_ref[pl.ds(c * _CB, _CB), :]
    return jax.lax.dot_general(
        a, zb, (((1,), (1,)), ((), ())),
        preferred_element_type=jnp.float32)


def _pass_n_kernel(z1_ref, z2_ref, out_ref):
    a = z1_ref[...]
    zb = z2_ref[pl.ds(pl.program_id(0) * _RB, _RB), :]
    d = jnp.sum(a * zb, axis=1)
    out_ref[...] = _lane_pack([
        (0, jnp.max(jnp.sum(a * a, axis=1))),
        (1, jnp.max(jnp.sum(zb * zb, axis=1))),
        (2, jnp.sum(_softplus(-d))),
        (3, jnp.sum((a - zb) ** 2)),
    ])[None]


def _pass_b_kernel(thr_ref, z1_ref, z2_ref, out_ref):
    a = z1_ref[...]
    ts = [thr_ref[0, j] for j in range(_NT)]

    def body(c, carry):
        s = _chunk_dot(a, z2_ref, c)
        return tuple(carry[j] + jnp.count_nonzero(s >= ts[j]).astype(jnp.int32)
                     for j in range(_NT))

    init = tuple(jnp.int32(0) for _ in range(_NT))
    res = jax.lax.fori_loop(0, _NC, body, init)
    pairs = [(j, res[j].astype(jnp.float32)) for j in range(_NT)]
    out_ref[...] = _lane_pack(pairs)[None]


def _pass_c_kernel(par_ref, z1_ref, z2_ref, out_ref):
    a = z1_ref[...]
    u_hi = par_ref[0, 0]
    l_lo = par_ref[0, 1]

    fast = jnp.logical_and(u_hi > 17.0, l_lo < -17.0)

    def make_body(use_exp):
        def body(c, carry):
            cnt_t, sum_t, cnt_b, sum_b = carry
            s = _chunk_dot(a, z2_ref, c)
            m_t = s > u_hi
            m_b = s < l_lo
            if use_exp:
                g = jnp.exp(-s)
                h = jnp.exp(s)
            else:
                g = _softplus(-s)
                h = _softplus(s)
            cnt_t = cnt_t + jnp.sum(jnp.where(m_t, 1.0, 0.0))
            sum_t = sum_t + jnp.sum(jnp.where(m_t, g, 0.0))
            cnt_b = cnt_b + jnp.sum(jnp.where(m_b, 1.0, 0.0))
            sum_b = sum_b + jnp.sum(jnp.where(m_b, h, 0.0))
            return cnt_t, sum_t, cnt_b, sum_b
        return body

    z = jnp.float32(0.0)
    init = (z, z, z, z)
    cnt_t, sum_t, cnt_b, sum_b = jax.lax.cond(
        fast,
        lambda: jax.lax.fori_loop(0, _NC, make_body(True), init),
        lambda: jax.lax.fori_loop(0, _NC, make_body(False), init),
    )
    out_ref[...] = _lane_pack([
        (0, cnt_t), (1, sum_t), (2, cnt_b), (3, sum_b),
    ])[None]


_Z1F_SPEC = pl.BlockSpec((_RB, _D), lambda i: (i, 0))
_Z2F_SPEC = pl.BlockSpec((_N, _D), lambda i: (0, 0))
_OUT_SPEC = pl.BlockSpec((1, 1, 128), lambda i: (i, 0, 0))
_OUT_SHAPE = jax.ShapeDtypeStruct((_G, 1, 128), jnp.float32)
_SMEM_SPEC = pl.BlockSpec(memory_space=pltpu.SMEM)


def kernel(epoch, z1, z2):
    del epoch
    z1 = z1.astype(jnp.float32)
    z2 = z2.astype(jnp.float32)
    z1b = z1.astype(jnp.bfloat16)
    z2b = z2.astype(jnp.bfloat16)

    pn = pl.pallas_call(
        _pass_n_kernel,
        grid=(_G,),
        in_specs=[_Z1F_SPEC, _Z2F_SPEC],
        out_specs=_OUT_SPEC,
        out_shape=_OUT_SHAPE,
    )(z1, z2)
    bound = jnp.sqrt(jnp.max(pn[:, 0, 0])) * jnp.sqrt(jnp.max(pn[:, 0, 1]))
    sum_diag = jnp.sum(pn[:, 0, 2])
    sum_sq = jnp.sum(pn[:, 0, 3])

    js = jnp.arange(1, _NT + 1, dtype=jnp.float32)
    thr = -bound + 2.0 * bound * js / (_NT + 1.0)
    thr_in = thr[None, :]

    pb = pl.pallas_call(
        _pass_b_kernel,
        grid=(_G,),
        in_specs=[_SMEM_SPEC, _Z1F_SPEC, _Z2F_SPEC],
        out_specs=_OUT_SPEC,
        out_shape=_OUT_SHAPE,
    )(thr_in, z1b, z2b)
    cnt_ge = jnp.sum(pb[:, 0, :_NT].astype(jnp.int32), axis=0)
    cnt_lt = jnp.int32(_N * _N) - cnt_ge

    ki = jnp.int32(_K)
    kf = jnp.float32(_K)
    m_hi = cnt_ge >= ki
    l_hi = jnp.maximum(-bound, jnp.max(jnp.where(m_hi, thr, -jnp.inf)))
    u_hi = jnp.minimum(bound, jnp.min(jnp.where(~m_hi, thr, jnp.inf)))
    m_lo = cnt_lt >= ki
    l_lo = jnp.maximum(-bound, jnp.max(jnp.where(~m_lo, thr, -jnp.inf)))
    u_lo = jnp.minimum(bound, jnp.min(jnp.where(m_lo, thr, jnp.inf)))

    par = jnp.stack([u_hi, l_lo])[None, :]

    pc = pl.pallas_call(
        _pass_c_kernel,
        grid=(_G,),
        in_specs=[_SMEM_SPEC, _Z1F_SPEC, _Z2F_SPEC],
        out_specs=_OUT_SPEC,
        out_shape=_OUT_SHAPE,
    )(par, z1b, z2b)
    pc = jnp.sum(pc[:, 0, :], axis=0)
    cnt_top, sum_top, cnt_bot, sum_bot = pc[0], pc[1], pc[2], pc[3]

    v_hi = 0.5 * (l_hi + u_hi)
    v_lo = 0.5 * (l_lo + u_lo)
    top_term = sum_top + (kf - cnt_top) * _softplus(-v_hi)
    bot_term = sum_bot + (kf - cnt_bot) * _softplus(v_lo)

    n_logits = jnp.float32(_N + 2 * _K)
    bce = (sum_diag + top_term + bot_term) / n_logits
    mse = sum_sq / jnp.float32(_N * _D)
    return bce + _LAM * mse * jnp.float32(_N)


# speedup vs baseline: 796.9627x; 1.2726x over previous
import jax
import jax.numpy as jnp
from jax.experimental import pallas as pl
from jax.experimental.pallas import tpu as pltpu

_N = 10000
_D = 128
_K = 1024
_LAM = 0.5
_RB = 400
_G = _N // _RB
_CB = 1000
_NC = _N // _CB
_NT = 3


def _lane_pack(pairs):
    lane = jax.lax.broadcasted_iota(jnp.int32, (1, 128), 1)
    row = jnp.zeros((1, 128), dtype=jnp.float32)
    for idx, val in pairs:
        row = jnp.where(lane == idx, val, row)
    return row


def _softplus(x):
    return jnp.maximum(x, 0.0) + jnp.log1p(jnp.exp(-jnp.abs(x)))


def _chunk_dot(a, z2_ref, c):
    zb = z2_ref[pl.ds(c * _CB, _CB), :]
    return jax.lax.dot_general(
        a, zb, (((1,), (1,)), ((), ())),
        preferred_element_type=jnp.float32)


def _pass_n_kernel(z1_ref, z2_ref, out_ref):
    a = z1_ref[...]
    zb = z2_ref[pl.ds(pl.program_id(0) * _RB, _RB), :]
    d = jnp.sum(a * zb, axis=1)
    out_ref[...] = _lane_pack([
        (0, jnp.max(jnp.sum(a * a, axis=1))),
        (1, jnp.max(jnp.sum(zb * zb, axis=1))),
        (2, jnp.sum(_softplus(-d))),
        (3, jnp.sum((a - zb) ** 2)),
    ])[None]


def _guarded_count(s, smn, smx, t):
    return jax.lax.cond(
        smn >= t,
        lambda: jnp.int32(_RB * _CB),
        lambda: jax.lax.cond(
            smx < t,
            lambda: jnp.int32(0),
            lambda: jnp.count_nonzero(s >= t).astype(jnp.int32)))


def _pass_b_kernel(thr_ref, z1_ref, z2_ref, out_ref):
    a = z1_ref[...]
    ts = [thr_ref[0, j] for j in range(_NT)]

    def body(c, carry):
        s = _chunk_dot(a, z2_ref, c)
        smn = jnp.min(s)
        smx = jnp.max(s)
        return tuple(carry[j] + _guarded_count(s, smn, smx, ts[j])
                     for j in range(_NT))

    init = tuple(jnp.int32(0) for _ in range(_NT))
    res = jax.lax.fori_loop(0, _NC, body, init)
    pairs = [(j, res[j].astype(jnp.float32)) for j in range(_NT)]
    out_ref[...] = _lane_pack(pairs)[None]


def _pass_c_kernel(par_ref, z1_ref, z2_ref, out_ref):
    a = z1_ref[...]
    u_hi = par_ref[0, 0]
    l_lo = par_ref[0, 1]

    fast = jnp.logical_and(u_hi > 17.0, l_lo < -17.0)

    def make_update(use_exp):
        def update(s, carry):
            cnt_t, sum_t, cnt_b, sum_b = carry
            m_t = s > u_hi
            m_b = s < l_lo
            if use_exp:
                g = jnp.exp(-s)
                h = jnp.exp(s)
            else:
                g = _softplus(-s)
                h = _softplus(s)
            cnt_t = cnt_t + jnp.sum(jnp.where(m_t, 1.0, 0.0))
            sum_t = sum_t + jnp.sum(jnp.where(m_t, g, 0.0))
            cnt_b = cnt_b + jnp.sum(jnp.where(m_b, 1.0, 0.0))
            sum_b = sum_b + jnp.sum(jnp.where(m_b, h, 0.0))
            return cnt_t, sum_t, cnt_b, sum_b
        return update

    def make_body(use_exp):
        update = make_update(use_exp)

        def body(c, carry):
            s = _chunk_dot(a, z2_ref, c)
            no_hit = jnp.logical_and(jnp.max(s) <= u_hi, jnp.min(s) >= l_lo)
            return jax.lax.cond(
                no_hit, lambda cr: cr, lambda cr: update(s, cr), carry)
        return body

    z = jnp.float32(0.0)
    init = (z, z, z, z)
    cnt_t, sum_t, cnt_b, sum_b = jax.lax.cond(
        fast,
        lambda: jax.lax.fori_loop(0, _NC, make_body(True), init),
        lambda: jax.lax.fori_loop(0, _NC, make_body(False), init),
    )
    out_ref[...] = _lane_pack([
        (0, cnt_t), (1, sum_t), (2, cnt_b), (3, sum_b),
    ])[None]


_Z1F_SPEC = pl.BlockSpec((_RB, _D), lambda i: (i, 0))
_Z2F_SPEC = pl.BlockSpec((_N, _D), lambda i: (0, 0))
_OUT_SPEC = pl.BlockSpec((1, 1, 128), lambda i: (i, 0, 0))
_OUT_SHAPE = jax.ShapeDtypeStruct((_G, 1, 128), jnp.float32)
_SMEM_SPEC = pl.BlockSpec(memory_space=pltpu.SMEM)


def kernel(epoch, z1, z2):
    del epoch
    z1 = z1.astype(jnp.float32)
    z2 = z2.astype(jnp.float32)
    z1b = z1.astype(jnp.bfloat16)
    z2b = z2.astype(jnp.bfloat16)

    pn = pl.pallas_call(
        _pass_n_kernel,
        grid=(_G,),
        in_specs=[_Z1F_SPEC, _Z2F_SPEC],
        out_specs=_OUT_SPEC,
        out_shape=_OUT_SHAPE,
    )(z1, z2)
    bound = jnp.sqrt(jnp.max(pn[:, 0, 0])) * jnp.sqrt(jnp.max(pn[:, 0, 1]))
    sum_diag = jnp.sum(pn[:, 0, 2])
    sum_sq = jnp.sum(pn[:, 0, 3])

    js = jnp.arange(1, _NT + 1, dtype=jnp.float32)
    thr = -bound + 2.0 * bound * js / (_NT + 1.0)
    thr_in = thr[None, :]

    pb = pl.pallas_call(
        _pass_b_kernel,
        grid=(_G,),
        in_specs=[_SMEM_SPEC, _Z1F_SPEC, _Z2F_SPEC],
        out_specs=_OUT_SPEC,
        out_shape=_OUT_SHAPE,
    )(thr_in, z1b, z2b)
    cnt_ge = jnp.sum(pb[:, 0, :_NT].astype(jnp.int32), axis=0)
    cnt_lt = jnp.int32(_N * _N) - cnt_ge

    ki = jnp.int32(_K)
    kf = jnp.float32(_K)
    m_hi = cnt_ge >= ki
    l_hi = jnp.maximum(-bound, jnp.max(jnp.where(m_hi, thr, -jnp.inf)))
    u_hi = jnp.minimum(bound, jnp.min(jnp.where(~m_hi, thr, jnp.inf)))
    m_lo = cnt_lt >= ki
    l_lo = jnp.maximum(-bound, jnp.max(jnp.where(~m_lo, thr, -jnp.inf)))
    u_lo = jnp.minimum(bound, jnp.min(jnp.where(m_lo, thr, jnp.inf)))

    par = jnp.stack([u_hi, l_lo])[None, :]

    pc = pl.pallas_call(
        _pass_c_kernel,
        grid=(_G,),
        in_specs=[_SMEM_SPEC, _Z1F_SPEC, _Z2F_SPEC],
        out_specs=_OUT_SPEC,
        out_shape=_OUT_SHAPE,
    )(par, z1b, z2b)
    pc = jnp.sum(pc[:, 0, :], axis=0)
    cnt_top, sum_top, cnt_bot, sum_bot = pc[0], pc[1], pc[2], pc[3]

    v_hi = 0.5 * (l_hi + u_hi)
    v_lo = 0.5 * (l_lo + u_lo)
    top_term = sum_top + (kf - cnt_top) * _softplus(-v_hi)
    bot_term = sum_bot + (kf - cnt_bot) * _softplus(v_lo)

    n_logits = jnp.float32(_N + 2 * _K)
    bce = (sum_diag + top_term + bot_term) / n_logits
    mse = sum_sq / jnp.float32(_N * _D)
    return bce + _LAM * mse * jnp.float32(_N)


# speedup vs baseline: 1035.1551x; 1.2989x over previous
import jax
import jax.numpy as jnp
from jax.experimental import pallas as pl
from jax.experimental.pallas import tpu as pltpu

_N = 10000
_D = 128
_K = 1024
_LAM = 0.5
_RB = 1000
_G = _N // _RB
_CB = 1000
_NC = _N // _CB
_NT = 3


def _lane_pack(pairs):
    lane = jax.lax.broadcasted_iota(jnp.int32, (1, 128), 1)
    row = jnp.zeros((1, 128), dtype=jnp.float32)
    for idx, val in pairs:
        row = jnp.where(lane == idx, val, row)
    return row


def _softplus(x):
    return jnp.maximum(x, 0.0) + jnp.log1p(jnp.exp(-jnp.abs(x)))


def _chunk_dot(a, z2_ref, c):
    zb = z2_ref[pl.ds(c * _CB, _CB), :]
    return jax.lax.dot_general(
        a, zb, (((1,), (1,)), ((), ())),
        preferred_element_type=jnp.float32)


def _pass_n_kernel(z1_ref, z2_ref, out_ref):
    a = z1_ref[...]
    zb = z2_ref[pl.ds(pl.program_id(0) * _RB, _RB), :]
    d = jnp.sum(a * zb, axis=1)
    out_ref[...] = _lane_pack([
        (0, jnp.max(jnp.sum(a * a, axis=1))),
        (1, jnp.max(jnp.sum(zb * zb, axis=1))),
        (2, jnp.sum(_softplus(-d))),
        (3, jnp.sum((a - zb) ** 2)),
    ])[None]


def _guarded_count(s, smn, smx, t):
    return jax.lax.cond(
        smn >= t,
        lambda: jnp.int32(_RB * _CB),
        lambda: jax.lax.cond(
            smx < t,
            lambda: jnp.int32(0),
            lambda: jnp.count_nonzero(s >= t).astype(jnp.int32)))


def _pass_b_kernel(thr_ref, z1_ref, z2_ref, out_ref):
    a = z1_ref[...]
    ts = [thr_ref[0, j] for j in range(_NT)]

    def body(c, carry):
        s = _chunk_dot(a, z2_ref, c)
        smn = jnp.min(s)
        smx = jnp.max(s)
        return tuple(carry[j] + _guarded_count(s, smn, smx, ts[j])
                     for j in range(_NT))

    init = tuple(jnp.int32(0) for _ in range(_NT))
    res = jax.lax.fori_loop(0, _NC, body, init)
    pairs = [(j, res[j].astype(jnp.float32)) for j in range(_NT)]
    out_ref[...] = _lane_pack(pairs)[None]


def _pass_c_kernel(par_ref, z1_ref, z2_ref, out_ref):
    a = z1_ref[...]
    u_hi = par_ref[0, 0]
    l_lo = par_ref[0, 1]

    fast = jnp.logical_and(u_hi > 17.0, l_lo < -17.0)

    def make_update(use_exp):
        def update(s, carry):
            cnt_t, sum_t, cnt_b, sum_b = carry
            m_t = s > u_hi
            m_b = s < l_lo
            if use_exp:
                g = jnp.exp(-s)
                h = jnp.exp(s)
            else:
                g = _softplus(-s)
                h = _softplus(s)
            cnt_t = cnt_t + jnp.sum(jnp.where(m_t, 1.0, 0.0))
            sum_t = sum_t + jnp.sum(jnp.where(m_t, g, 0.0))
            cnt_b = cnt_b + jnp.sum(jnp.where(m_b, 1.0, 0.0))
            sum_b = sum_b + jnp.sum(jnp.where(m_b, h, 0.0))
            return cnt_t, sum_t, cnt_b, sum_b
        return update

    def make_body(use_exp):
        update = make_update(use_exp)

        def body(c, carry):
            s = _chunk_dot(a, z2_ref, c)
            no_hit = jnp.logical_and(jnp.max(s) <= u_hi, jnp.min(s) >= l_lo)
            return jax.lax.cond(
                no_hit, lambda cr: cr, lambda cr: update(s, cr), carry)
        return body

    z = jnp.float32(0.0)
    init = (z, z, z, z)
    cnt_t, sum_t, cnt_b, sum_b = jax.lax.cond(
        fast,
        lambda: jax.lax.fori_loop(0, _NC, make_body(True), init),
        lambda: jax.lax.fori_loop(0, _NC, make_body(False), init),
    )
    out_ref[...] = _lane_pack([
        (0, cnt_t), (1, sum_t), (2, cnt_b), (3, sum_b),
    ])[None]


_Z1F_SPEC = pl.BlockSpec((_RB, _D), lambda i: (i, 0))
_Z2F_SPEC = pl.BlockSpec((_N, _D), lambda i: (0, 0))
_OUT_SPEC = pl.BlockSpec((1, 1, 128), lambda i: (i, 0, 0))
_OUT_SHAPE = jax.ShapeDtypeStruct((_G, 1, 128), jnp.float32)
_SMEM_SPEC = pl.BlockSpec(memory_space=pltpu.SMEM)


def kernel(epoch, z1, z2):
    del epoch
    z1 = z1.astype(jnp.float32)
    z2 = z2.astype(jnp.float32)
    z1b = z1.astype(jnp.bfloat16)
    z2b = z2.astype(jnp.bfloat16)

    pn = pl.pallas_call(
        _pass_n_kernel,
        grid=(_G,),
        in_specs=[_Z1F_SPEC, _Z2F_SPEC],
        out_specs=_OUT_SPEC,
        out_shape=_OUT_SHAPE,
    )(z1, z2)
    bound = jnp.sqrt(jnp.max(pn[:, 0, 0])) * jnp.sqrt(jnp.max(pn[:, 0, 1]))
    sum_diag = jnp.sum(pn[:, 0, 2])
    sum_sq = jnp.sum(pn[:, 0, 3])

    js = jnp.arange(1, _NT + 1, dtype=jnp.float32)
    thr = -bound + 2.0 * bound * js / (_NT + 1.0)
    thr_in = thr[None, :]

    pb = pl.pallas_call(
        _pass_b_kernel,
        grid=(_G,),
        in_specs=[_SMEM_SPEC, _Z1F_SPEC, _Z2F_SPEC],
        out_specs=_OUT_SPEC,
        out_shape=_OUT_SHAPE,
    )(thr_in, z1b, z2b)
    cnt_ge = jnp.sum(pb[:, 0, :_NT].astype(jnp.int32), axis=0)
    cnt_lt = jnp.int32(_N * _N) - cnt_ge

    ki = jnp.int32(_K)
    kf = jnp.float32(_K)
    m_hi = cnt_ge >= ki
    l_hi = jnp.maximum(-bound, jnp.max(jnp.where(m_hi, thr, -jnp.inf)))
    u_hi = jnp.minimum(bound, jnp.min(jnp.where(~m_hi, thr, jnp.inf)))
    m_lo = cnt_lt >= ki
    l_lo = jnp.maximum(-bound, jnp.max(jnp.where(~m_lo, thr, -jnp.inf)))
    u_lo = jnp.minimum(bound, jnp.min(jnp.where(m_lo, thr, jnp.inf)))

    par = jnp.stack([u_hi, l_lo])[None, :]

    pc = pl.pallas_call(
        _pass_c_kernel,
        grid=(_G,),
        in_specs=[_SMEM_SPEC, _Z1F_SPEC, _Z2F_SPEC],
        out_specs=_OUT_SPEC,
        out_shape=_OUT_SHAPE,
    )(par, z1b, z2b)
    pc = jnp.sum(pc[:, 0, :], axis=0)
    cnt_top, sum_top, cnt_bot, sum_bot = pc[0], pc[1], pc[2], pc[3]

    v_hi = 0.5 * (l_hi + u_hi)
    v_lo = 0.5 * (l_lo + u_lo)
    top_term = sum_top + (kf - cnt_top) * _softplus(-v_hi)
    bot_term = sum_bot + (kf - cnt_bot) * _softplus(v_lo)

    n_logits = jnp.float32(_N + 2 * _K)
    bce = (sum_diag + top_term + bot_term) / n_logits
    mse = sum_sq / jnp.float32(_N * _D)
    return bce + _LAM * mse * jnp.float32(_N)


# speedup vs baseline: 1557.4360x; 1.5045x over previous
import jax
import jax.numpy as jnp
from jax.experimental import pallas as pl
from jax.experimental.pallas import tpu as pltpu

_N = 10000
_D = 128
_K = 1024
_LAM = 0.5
_RB = 1000
_G = _N // _RB
_CB = 1000
_NC = _N // _CB
_NT = 3


def _lane_pack(pairs):
    lane = jax.lax.broadcasted_iota(jnp.int32, (1, 128), 1)
    row = jnp.zeros((1, 128), dtype=jnp.float32)
    for idx, val in pairs:
        row = jnp.where(lane == idx, val, row)
    return row


def _softplus(x):
    return jnp.maximum(x, 0.0) + jnp.log1p(jnp.exp(-jnp.abs(x)))


def _chunk_dot(a, z2_ref, c):
    zb = z2_ref[pl.ds(c * _CB, _CB), :]
    return jax.lax.dot_general(
        a, zb, (((1,), (1,)), ((), ())),
        preferred_element_type=jnp.float32)


def _pass_n_kernel(z1_ref, z2_ref, out_ref):
    a = z1_ref[...]
    zb = z2_ref[pl.ds(pl.program_id(0) * _RB, _RB), :]
    d = jnp.sum(a * zb, axis=1)
    out_ref[...] = _lane_pack([
        (0, jnp.max(jnp.sum(a * a, axis=1))),
        (1, jnp.max(jnp.sum(zb * zb, axis=1))),
        (2, jnp.sum(_softplus(-d))),
        (3, jnp.sum((a - zb) ** 2)),
    ])[None]


def _guarded_count(s, smn, smx, t):
    return jax.lax.cond(
        smn >= t,
        lambda: jnp.int32(_RB * _CB),
        lambda: jax.lax.cond(
            smx < t,
            lambda: jnp.int32(0),
            lambda: jnp.count_nonzero(s >= t).astype(jnp.int32)))


def _pass_bc_kernel(thr_ref, z1_ref, z2_ref, out_ref):
    a = z1_ref[...]
    ts = [thr_ref[0, j] for j in range(_NT)]
    t_lo, t_hi = ts[0], ts[_NT - 1]

    def body(c, carry):
        c0, c1, c2, cnt_t, sum_t, cnt_b, sum_b = carry
        s = _chunk_dot(a, z2_ref, c)
        smn = jnp.min(s)
        smx = jnp.max(s)
        c0 = c0 + _guarded_count(s, smn, smx, ts[0])
        c1 = c1 + _guarded_count(s, smn, smx, ts[1])
        c2 = c2 + _guarded_count(s, smn, smx, ts[2])

        def top_update(cr):
            ct, st = cr
            m = s > t_hi
            return (ct + jnp.sum(jnp.where(m, 1.0, 0.0)),
                    st + jnp.sum(jnp.where(m, jnp.exp(-s), 0.0)))

        def bot_update(cr):
            cb, sb = cr
            m = s < t_lo
            return (cb + jnp.sum(jnp.where(m, 1.0, 0.0)),
                    sb + jnp.sum(jnp.where(m, jnp.exp(s), 0.0)))

        cnt_t, sum_t = jax.lax.cond(
            smx <= t_hi, lambda cr: cr, top_update, (cnt_t, sum_t))
        cnt_b, sum_b = jax.lax.cond(
            smn >= t_lo, lambda cr: cr, bot_update, (cnt_b, sum_b))
        return c0, c1, c2, cnt_t, sum_t, cnt_b, sum_b

    zi = jnp.int32(0)
    zf = jnp.float32(0.0)
    res = jax.lax.fori_loop(0, _NC, body, (zi, zi, zi, zf, zf, zf, zf))
    pairs = [(j, res[j].astype(jnp.float32)) for j in range(_NT)]
    pairs += [(_NT + j, res[_NT + j]) for j in range(4)]
    out_ref[...] = _lane_pack(pairs)[None]


def _pass_c_kernel(par_ref, z1_ref, z2_ref, out_ref):
    a = z1_ref[...]
    u_hi = par_ref[0, 0]
    l_lo = par_ref[0, 1]

    fast = jnp.logical_and(u_hi > 17.0, l_lo < -17.0)

    def make_update(use_exp):
        def update(s, carry):
            cnt_t, sum_t, cnt_b, sum_b = carry
            m_t = s > u_hi
            m_b = s < l_lo
            if use_exp:
                g = jnp.exp(-s)
                h = jnp.exp(s)
            else:
                g = _softplus(-s)
                h = _softplus(s)
            cnt_t = cnt_t + jnp.sum(jnp.where(m_t, 1.0, 0.0))
            sum_t = sum_t + jnp.sum(jnp.where(m_t, g, 0.0))
            cnt_b = cnt_b + jnp.sum(jnp.where(m_b, 1.0, 0.0))
            sum_b = sum_b + jnp.sum(jnp.where(m_b, h, 0.0))
            return cnt_t, sum_t, cnt_b, sum_b
        return update

    def make_body(use_exp):
        update = make_update(use_exp)

        def body(c, carry):
            s = _chunk_dot(a, z2_ref, c)
            no_hit = jnp.logical_and(jnp.max(s) <= u_hi, jnp.min(s) >= l_lo)
            return jax.lax.cond(
                no_hit, lambda cr: cr, lambda cr: update(s, cr), carry)
        return body

    z = jnp.float32(0.0)
    init = (z, z, z, z)
    cnt_t, sum_t, cnt_b, sum_b = jax.lax.cond(
        fast,
        lambda: jax.lax.fori_loop(0, _NC, make_body(True), init),
        lambda: jax.lax.fori_loop(0, _NC, make_body(False), init),
    )
    out_ref[...] = _lane_pack([
        (0, cnt_t), (1, sum_t), (2, cnt_b), (3, sum_b),
    ])[None]


_Z1F_SPEC = pl.BlockSpec((_RB, _D), lambda i: (i, 0))
_Z2F_SPEC = pl.BlockSpec((_N, _D), lambda i: (0, 0))
_OUT_SPEC = pl.BlockSpec((1, 1, 128), lambda i: (i, 0, 0))
_OUT_SHAPE = jax.ShapeDtypeStruct((_G, 1, 128), jnp.float32)
_SMEM_SPEC = pl.BlockSpec(memory_space=pltpu.SMEM)


def kernel(epoch, z1, z2):
    del epoch
    z1 = z1.astype(jnp.float32)
    z2 = z2.astype(jnp.float32)
    z1b = z1.astype(jnp.bfloat16)
    z2b = z2.astype(jnp.bfloat16)

    pn = pl.pallas_call(
        _pass_n_kernel,
        grid=(_G,),
        in_specs=[_Z1F_SPEC, _Z2F_SPEC],
        out_specs=_OUT_SPEC,
        out_shape=_OUT_SHAPE,
    )(z1, z2)
    bound = jnp.sqrt(jnp.max(pn[:, 0, 0])) * jnp.sqrt(jnp.max(pn[:, 0, 1]))
    sum_diag = jnp.sum(pn[:, 0, 2])
    sum_sq = jnp.sum(pn[:, 0, 3])

    js = jnp.arange(1, _NT + 1, dtype=jnp.float32)
    thr = -bound + 2.0 * bound * js / (_NT + 1.0)
    thr_in = thr[None, :]

    pb = pl.pallas_call(
        _pass_bc_kernel,
        grid=(_G,),
        in_specs=[_SMEM_SPEC, _Z1F_SPEC, _Z2F_SPEC],
        out_specs=_OUT_SPEC,
        out_shape=_OUT_SHAPE,
    )(thr_in, z1b, z2b)
    cnt_ge = jnp.sum(pb[:, 0, :_NT].astype(jnp.int32), axis=0)
    cnt_lt = jnp.int32(_N * _N) - cnt_ge
    mg = jnp.sum(pb[:, 0, _NT:_NT + 4], axis=0)

    ki = jnp.int32(_K)
    kf = jnp.float32(_K)
    m_hi = cnt_ge >= ki
    l_hi = jnp.maximum(-bound, jnp.max(jnp.where(m_hi, thr, -jnp.inf)))
    u_hi = jnp.minimum(bound, jnp.min(jnp.where(~m_hi, thr, jnp.inf)))
    m_lo = cnt_lt >= ki
    l_lo = jnp.maximum(-bound, jnp.max(jnp.where(~m_lo, thr, -jnp.inf)))
    u_lo = jnp.minimum(bound, jnp.min(jnp.where(m_lo, thr, jnp.inf)))

    use_merged = ((u_hi == thr[_NT - 1]) & (l_lo == thr[0])
                  & (u_hi > 17.0) & (l_lo < -17.0))

    def run_pass_c(_):
        par = jnp.stack([u_hi, l_lo])[None, :]
        pc = pl.pallas_call(
            _pass_c_kernel,
            grid=(_G,),
            in_specs=[_SMEM_SPEC, _Z1F_SPEC, _Z2F_SPEC],
            out_specs=_OUT_SPEC,
            out_shape=_OUT_SHAPE,
        )(par, z1b, z2b)
        return jnp.sum(pc[:, 0, :4], axis=0)

    pc = jax.lax.cond(use_merged, lambda _: mg, run_pass_c, 0)
    cnt_top, sum_top, cnt_bot, sum_bot = pc[0], pc[1], pc[2], pc[3]

    v_hi = 0.5 * (l_hi + u_hi)
    v_lo = 0.5 * (l_lo + u_lo)
    top_term = sum_top + (kf - cnt_top) * _softplus(-v_hi)
    bot_term = sum_bot + (kf - cnt_bot) * _softplus(v_lo)

    n_logits = jnp.float32(_N + 2 * _K)
    bce = (sum_diag + top_term + bot_term) / n_logits
    mse = sum_sq / jnp.float32(_N * _D)
    return bce + _LAM * mse * jnp.float32(_N)


# speedup vs baseline: 1709.9277x; 1.0979x over previous
import jax
import jax.numpy as jnp
from jax.experimental import pallas as pl
from jax.experimental.pallas import tpu as pltpu

_N = 10000
_D = 128
_K = 1024
_LAM = 0.5
_RB = 2000
_G = _N // _RB
_CB = 1000
_NC = _N // _CB
_NT = 3


def _lane_pack(pairs):
    lane = jax.lax.broadcasted_iota(jnp.int32, (1, 128), 1)
    row = jnp.zeros((1, 128), dtype=jnp.float32)
    for idx, val in pairs:
        row = jnp.where(lane == idx, val, row)
    return row


def _softplus(x):
    return jnp.maximum(x, 0.0) + jnp.log1p(jnp.exp(-jnp.abs(x)))


def _chunk_dot(a, z2_ref, c):
    zb = z2_ref[pl.ds(c * _CB, _CB), :]
    return jax.lax.dot_general(
        a, zb, (((1,), (1,)), ((), ())),
        preferred_element_type=jnp.float32)


def _pass_n_kernel(z1_ref, z2_ref, out_ref):
    a = z1_ref[...]
    zb = z2_ref[pl.ds(pl.program_id(0) * _RB, _RB), :]
    d = jnp.sum(a * zb, axis=1)
    out_ref[...] = _lane_pack([
        (0, jnp.max(jnp.sum(a * a, axis=1))),
        (1, jnp.max(jnp.sum(zb * zb, axis=1))),
        (2, jnp.sum(_softplus(-d))),
        (3, jnp.sum((a - zb) ** 2)),
    ])[None]


def _guarded_count(s, smn, smx, t):
    return jax.lax.cond(
        smn >= t,
        lambda: jnp.int32(_RB * _CB),
        lambda: jax.lax.cond(
            smx < t,
            lambda: jnp.int32(0),
            lambda: jnp.count_nonzero(s >= t).astype(jnp.int32)))


def _pass_bc_kernel(thr_ref, z1_ref, z2_ref, out_ref):
    a = z1_ref[...]
    ts = [thr_ref[0, j] for j in range(_NT)]
    t_lo, t_hi = ts[0], ts[_NT - 1]

    def body(c, carry):
        c0, c1, c2, cnt_t, sum_t, cnt_b, sum_b = carry
        s = _chunk_dot(a, z2_ref, c)
        smn = jnp.min(s)
        smx = jnp.max(s)
        c0 = c0 + _guarded_count(s, smn, smx, ts[0])
        sign = jax.lax.shift_right_logical(
            jax.lax.bitcast_convert_type(s, jnp.int32), jnp.int32(31))
        c1 = c1 + jnp.int32(_RB * _CB) - jnp.sum(sign)
        c2 = c2 + _guarded_count(s, smn, smx, ts[2])

        def top_update(cr):
            ct, st = cr
            m = s > t_hi
            return (ct + jnp.sum(jnp.where(m, 1.0, 0.0)),
                    st + jnp.sum(jnp.where(m, jnp.exp(-s), 0.0)))

        def bot_update(cr):
            cb, sb = cr
            m = s < t_lo
            return (cb + jnp.sum(jnp.where(m, 1.0, 0.0)),
                    sb + jnp.sum(jnp.where(m, jnp.exp(s), 0.0)))

        cnt_t, sum_t = jax.lax.cond(
            smx <= t_hi, lambda cr: cr, top_update, (cnt_t, sum_t))
        cnt_b, sum_b = jax.lax.cond(
            smn >= t_lo, lambda cr: cr, bot_update, (cnt_b, sum_b))
        return c0, c1, c2, cnt_t, sum_t, cnt_b, sum_b

    zi = jnp.int32(0)
    zf = jnp.float32(0.0)
    res = jax.lax.fori_loop(0, _NC, body, (zi, zi, zi, zf, zf, zf, zf),
                            unroll=2)
    pairs = [(j, res[j].astype(jnp.float32)) for j in range(_NT)]
    pairs += [(_NT + j, res[_NT + j]) for j in range(4)]
    out_ref[...] = _lane_pack(pairs)[None]


def _pass_c_kernel(par_ref, z1_ref, z2_ref, out_ref):
    a = z1_ref[...]
    u_hi = par_ref[0, 0]
    l_lo = par_ref[0, 1]

    fast = jnp.logical_and(u_hi > 17.0, l_lo < -17.0)

    def make_update(use_exp):
        def update(s, carry):
            cnt_t, sum_t, cnt_b, sum_b = carry
            m_t = s > u_hi
            m_b = s < l_lo
            if use_exp:
                g = jnp.exp(-s)
                h = jnp.exp(s)
            else:
                g = _softplus(-s)
                h = _softplus(s)
            cnt_t = cnt_t + jnp.sum(jnp.where(m_t, 1.0, 0.0))
            sum_t = sum_t + jnp.sum(jnp.where(m_t, g, 0.0))
            cnt_b = cnt_b + jnp.sum(jnp.where(m_b, 1.0, 0.0))
            sum_b = sum_b + jnp.sum(jnp.where(m_b, h, 0.0))
            return cnt_t, sum_t, cnt_b, sum_b
        return update

    def make_body(use_exp):
        update = make_update(use_exp)

        def body(c, carry):
            s = _chunk_dot(a, z2_ref, c)
            no_hit = jnp.logical_and(jnp.max(s) <= u_hi, jnp.min(s) >= l_lo)
            return jax.lax.cond(
                no_hit, lambda cr: cr, lambda cr: update(s, cr), carry)
        return body

    z = jnp.float32(0.0)
    init = (z, z, z, z)
    cnt_t, sum_t, cnt_b, sum_b = jax.lax.cond(
        fast,
        lambda: jax.lax.fori_loop(0, _NC, make_body(True), init),
        lambda: jax.lax.fori_loop(0, _NC, make_body(False), init),
    )
    out_ref[...] = _lane_pack([
        (0, cnt_t), (1, sum_t), (2, cnt_b), (3, sum_b),
    ])[None]


_Z1F_SPEC = pl.BlockSpec((_RB, _D), lambda i: (i, 0))
_Z2F_SPEC = pl.BlockSpec((_N, _D), lambda i: (0, 0))
_OUT_SPEC = pl.BlockSpec((1, 1, 128), lambda i: (i, 0, 0))
_OUT_SHAPE = jax.ShapeDtypeStruct((_G, 1, 128), jnp.float32)
_SMEM_SPEC = pl.BlockSpec(memory_space=pltpu.SMEM)


def kernel(epoch, z1, z2):
    del epoch
    z1 = z1.astype(jnp.float32)
    z2 = z2.astype(jnp.float32)
    z1b = z1.astype(jnp.bfloat16)
    z2b = z2.astype(jnp.bfloat16)

    pn = pl.pallas_call(
        _pass_n_kernel,
        grid=(_G,),
        in_specs=[_Z1F_SPEC, _Z2F_SPEC],
        out_specs=_OUT_SPEC,
        out_shape=_OUT_SHAPE,
    )(z1, z2)
    bound = jnp.sqrt(jnp.max(pn[:, 0, 0])) * jnp.sqrt(jnp.max(pn[:, 0, 1]))
    sum_diag = jnp.sum(pn[:, 0, 2])
    sum_sq = jnp.sum(pn[:, 0, 3])

    js = jnp.arange(1, _NT + 1, dtype=jnp.float32)
    thr = -bound + 2.0 * bound * js / (_NT + 1.0)
    thr_in = thr[None, :]

    pb = pl.pallas_call(
        _pass_bc_kernel,
        grid=(_G,),
        in_specs=[_SMEM_SPEC, _Z1F_SPEC, _Z2F_SPEC],
        out_specs=_OUT_SPEC,
        out_shape=_OUT_SHAPE,
    )(thr_in, z1b, z2b)
    cnt_ge = jnp.sum(pb[:, 0, :_NT].astype(jnp.int32), axis=0)
    cnt_lt = jnp.int32(_N * _N) - cnt_ge
    mg = jnp.sum(pb[:, 0, _NT:_NT + 4], axis=0)

    ki = jnp.int32(_K)
    kf = jnp.float32(_K)
    m_hi = cnt_ge >= ki
    l_hi = jnp.maximum(-bound, jnp.max(jnp.where(m_hi, thr, -jnp.inf)))
    u_hi = jnp.minimum(bound, jnp.min(jnp.where(~m_hi, thr, jnp.inf)))
    m_lo = cnt_lt >= ki
    l_lo = jnp.maximum(-bound, jnp.max(jnp.where(~m_lo, thr, -jnp.inf)))
    u_lo = jnp.minimum(bound, jnp.min(jnp.where(m_lo, thr, jnp.inf)))

    use_merged = ((u_hi == thr[_NT - 1]) & (l_lo == thr[0])
                  & (u_hi > 17.0) & (l_lo < -17.0))

    def run_pass_c(_):
        par = jnp.stack([u_hi, l_lo])[None, :]
        pc = pl.pallas_call(
            _pass_c_kernel,
            grid=(_G,),
            in_specs=[_SMEM_SPEC, _Z1F_SPEC, _Z2F_SPEC],
            out_specs=_OUT_SPEC,
            out_shape=_OUT_SHAPE,
        )(par, z1b, z2b)
        return jnp.sum(pc[:, 0, :4], axis=0)

    pc = jax.lax.cond(use_merged, lambda _: mg, run_pass_c, 0)
    cnt_top, sum_top, cnt_bot, sum_bot = pc[0], pc[1], pc[2], pc[3]

    v_hi = 0.5 * (l_hi + u_hi)
    v_lo = 0.5 * (l_lo + u_lo)
    top_term = sum_top + (kf - cnt_top) * _softplus(-v_hi)
    bot_term = sum_bot + (kf - cnt_bot) * _softplus(v_lo)

    n_logits = jnp.float32(_N + 2 * _K)
    bce = (sum_diag + top_term + bot_term) / n_logits
    mse = sum_sq / jnp.float32(_N * _D)
    return bce + _LAM * mse * jnp.float32(_N)
